# Initial kernel scaffold; baseline (speedup 1.0000x reference)
import jax
import jax.numpy as jnp
from jax.experimental import pallas as pl


def kernel(x, pe_table):
    raise NotImplementedError("write your pallas kernel here")


# speedup vs baseline: 1.7230x; 1.7230x over previous
import functools

import jax
import jax.numpy as jnp
from jax.experimental import pallas as pl


def _pe_add_block(x_ref, pe_ref, o_ref):
    o_ref[...] = x_ref[...] + pe_ref[...][None, :, :]


@functools.partial(jax.jit, static_argnames=("block_s",))
def _pe_add(x, pe, block_s=512):
    B, S, D = x.shape
    grid = (S // block_s,)
    return pl.pallas_call(
        _pe_add_block,
        grid=grid,
        in_specs=[
            pl.BlockSpec((B, block_s, D), lambda s: (0, s, 0)),
            pl.BlockSpec((block_s, D), lambda s: (s, 0)),
        ],
        out_specs=pl.BlockSpec((B, block_s, D), lambda s: (0, s, 0)),
        out_shape=jax.ShapeDtypeStruct((B, S, D), x.dtype),
    )(x, pe)


def kernel(x, pe_table):
    S_cur = x.shape[1]
    return _pe_add(x, pe_table[:S_cur])
